# Initial kernel scaffold; baseline (speedup 1.0000x reference)
import jax
import jax.numpy as jnp
from jax.experimental import pallas as pl


def kernel(points):
    raise NotImplementedError("write your pallas kernel here")


# speedup vs baseline: 5.3989x; 5.3989x over previous
import jax
import jax.numpy as jnp
from jax import lax
from jax.experimental import pallas as pl
from jax.experimental.pallas import tpu as pltpu
from jax.experimental.pallas import tpu_sc as plsc

N = 200000
C = 4
NW = 16
CH = 12544
NPAD = NW * CH
NWIN = 16
WROWS = CH // NWIN
WVECS = WROWS // 16
VC = 8
NBIN = 25000
RB = 1664
BINS_PAD = NW * RB
RVECS = RB // 16
SENT = BINS_PAD - 1
WPB = BINS_PAD // 4
RW = RB // 4
RWVECS = RW // 16
PLANE = WPB
SAT = 7
RSH = 14
RMASK = (1 << RSH) - 1
MAXP = 5
MAXV = 16000
VOX_ROWS = MAXV * MAXP
VOX_TOT = 80800
VSTRIPE = VOX_TOT // NW
NV_TOT = 17664
NVSTRIPE = NV_TOT // NW


def _body(pts, vox, nump, coors, histw_hbm,
          counter, histw, densebuf, win, packed, totals, vvtmp, sums2,
          numstage, densestage, coorstage, idxstage, ptstage,
          dense_sh, sums_sh, sem):
  wid = lax.axis_index("s")
  iota = lax.iota(jnp.int32, 16)
  ones = jnp.ones((16,), jnp.int32)
  zeros = jnp.zeros((16,), jnp.int32)

  def _z_pt(i, c):
    plsc.store_scatter(ptstage, [(i * 2) + (iota >> 3), iota & 7], zeros)
    return c
  lax.fori_loop(0, WROWS * VC // 16, _z_pt, 0)

  def _z_coor(i, c):
    plsc.store_scatter(coorstage, [(i * 2) + (iota >> 3), iota & 7], zeros)
    return c
  lax.fori_loop(0, NVSTRIPE * VC // 16, _z_coor, 0)

  def _z_num(i, c):
    numstage[pl.ds(i * 16, 16)] = zeros
    return c
  lax.fori_loop(0, NVSTRIPE // 16, _z_num, 0)

  def _z_cnt(i, c):
    counter[pl.ds(i * 16, 16)] = zeros
    return c
  lax.fori_loop(0, BINS_PAD // 16, _z_cnt, 0)

  vr0 = wid * VSTRIPE
  for k in range(VSTRIPE // WROWS):
    pltpu.sync_copy(ptstage.at[pl.ds(0, WROWS), :],
                    vox.at[pl.ds(vr0 + k * WROWS, WROWS), :])
  rem = VSTRIPE - (VSTRIPE // WROWS) * WROWS
  pltpu.sync_copy(ptstage.at[pl.ds(0, rem), :],
                  vox.at[pl.ds(vr0 + VSTRIPE - rem, rem), :])
  nr0 = wid * NVSTRIPE
  pltpu.sync_copy(numstage.at[pl.ds(0, NVSTRIPE)],
                  nump.at[pl.ds(nr0, NVSTRIPE)])
  pltpu.sync_copy(coorstage.at[pl.ds(0, NVSTRIPE), :],
                  coors.at[pl.ds(nr0, NVSTRIPE), :])

  vs_xy = jnp.float32(0.02)
  vs_z = jnp.float32(0.1)

  def _p1_win(w, c0):
    pltpu.sync_copy(pts.at[pl.ds(wid * CH + w * WROWS, WROWS), :], win)

    def _p1_vec(j, c1):
      r = j * 16 + iota
      gx = plsc.load_gather(win, [r, zeros])
      gy = plsc.load_gather(win, [r, zeros + 1])
      gz = plsc.load_gather(win, [r, zeros + 2])
      vx = (gx / vs_xy).astype(jnp.int32)
      vy = (gy / vs_xy).astype(jnp.int32)
      vz = (gz / vs_z).astype(jnp.int32)
      valid = ((vx >= 0) & (vx < 50) & (vy >= 0) & (vy < 50)
               & (vz >= 0) & (vz < 10))
      binv = jnp.where(valid, vz * 2500 + vy * 50 + vx, SENT)
      cnt, lastm = plsc.scan_count(binv)
      old = plsc.load_gather(counter, [binv])
      plsc.store_scatter(counter, [binv], jnp.minimum(old + cnt, SAT),
                         mask=lastm)
      packed[pl.ds((w * WVECS + j) * 16, 16)] = (
          binv * (1 << RSH) + (old + cnt - 1))
      return c1

    lax.fori_loop(0, WVECS, _p1_vec, 0)
    return c0

  lax.fori_loop(0, NWIN, _p1_win, 0)

  def _pack(i, c):
    k = (i * 16 + iota) * 4
    h0 = plsc.load_gather(counter, [k])
    h1 = plsc.load_gather(counter, [k + 1])
    h2 = plsc.load_gather(counter, [k + 2])
    h3 = plsc.load_gather(counter, [k + 3])
    histw[pl.ds(i * 16, 16)] = h0 | (h1 << 8) | (h2 << 16) | (h3 << 24)
    return c

  lax.fori_loop(0, WPB // 16, _pack, 0)
  pltpu.sync_copy(histw.at[pl.ds(0, WPB)],
                  histw_hbm.at[pl.ds(wid * WPB, WPB)])
  plsc.subcore_barrier()

  w0 = wid * RW
  for wp in range(NW):
    pltpu.sync_copy(histw_hbm.at[pl.ds(wp * WPB + w0, RW)],
                    histw.at[pl.ds(wp * RW, RW)])

  def _p2a(i, c):
    acc0 = zeros
    acc1 = zeros
    acc2 = zeros
    acc3 = zeros
    for wp in range(NW):
      wv = histw[pl.ds(wp * RW + i * 16, 16)]
      h0 = wv & 255
      h1 = (wv >> 8) & 255
      h2 = (wv >> 16) & 255
      h3 = (wv >> 24) & 255
      histw[pl.ds(wp * RW + i * 16, 16)] = (
          acc0 | (acc1 << 8) | (acc2 << 16) | (acc3 << 24))
      acc0 = acc0 + h0
      acc1 = acc1 + h1
      acc2 = acc2 + h2
      acc3 = acc3 + h3
    k = (i * 16 + iota) * 4
    plsc.store_scatter(totals, [k], acc0)
    plsc.store_scatter(totals, [k + 1], acc1)
    plsc.store_scatter(totals, [k + 2], acc2)
    plsc.store_scatter(totals, [k + 3], acc3)
    return c

  lax.fori_loop(0, RWVECS, _p2a, 0)
  for wp in range(NW):
    pltpu.sync_copy(histw.at[pl.ds(wp * RW, RW)],
                    histw_hbm.at[pl.ds(wp * WPB + w0, RW)])

  col0 = wid * RB

  def _p2b(j, carry):
    tv = totals[pl.ds(j * 16, 16)]
    binv = col0 + j * 16 + iota
    occ = (tv > 0) & (binv < NBIN)
    cinc = plsc.cumsum(ones, mask=occ) + carry
    vvtmp[pl.ds(j * 16, 16)] = cinc
    return jnp.max(cinc)

  range_occ = lax.fori_loop(0, RVECS, _p2b, jnp.int32(0))
  sums2[pl.ds(0, 16)] = zeros + range_occ
  pltpu.sync_copy(sums2.at[pl.ds(0, 16)], sums_sh.at[pl.ds(wid * 16, 16)])
  plsc.subcore_barrier()

  pltpu.sync_copy(sums_sh, sums2)
  svec = plsc.load_gather(sums2, [iota * 16 + iota])
  base = jnp.sum(jnp.where(iota < wid, svec, 0))

  def _pref_d(i, c):
    densestage[pl.ds(i * 16, 16)] = MAXV + i * 16 + iota
    return c

  lax.fori_loop(0, RVECS, _pref_d, 0)

  def _p2d(j, nk):
    cinc = vvtmp[pl.ds(j * 16, 16)]
    tv = totals[pl.ds(j * 16, 16)]
    lbl = j * 16 + iota
    binv = col0 + lbl
    occ = (tv > 0) & (binv < NBIN)
    dnv = base + cinc - occ.astype(jnp.int32)
    plsc.store_scatter(histw, [(lbl & 3) * RW + (lbl >> 2)], dnv)
    keep = occ & (dnv < MAXV)
    pos = plsc.cumsum(ones, mask=keep) - 1 + nk
    numv = jnp.minimum(tv, MAXP)
    z = lax.div(binv, 2500)
    rm = binv - z * 2500
    y = lax.div(rm, 50)
    x = rm - y * 50
    plsc.store_scatter(numstage, [pos], numv, mask=keep)
    plsc.store_scatter(densestage, [pos], dnv, mask=keep)
    plsc.store_scatter(coorstage, [pos, zeros], zeros, mask=keep)
    plsc.store_scatter(coorstage, [pos, zeros + 1], z, mask=keep)
    plsc.store_scatter(coorstage, [pos, zeros + 2], y, mask=keep)
    plsc.store_scatter(coorstage, [pos, zeros + 3], x, mask=keep)
    return nk + plsc.all_reduce_population_count(keep)

  lax.fori_loop(0, RVECS, _p2d, zeros)
  for p in range(4):
    pltpu.sync_copy(histw.at[pl.ds(p * RW, RW)],
                    dense_sh.at[pl.ds(p * PLANE + wid * RW, RW)])
  pltpu.async_copy(numstage, nump.at[densestage], sem).wait()
  pltpu.async_copy(coorstage, coors.at[densestage], sem).wait()
  plsc.subcore_barrier()

  pltpu.sync_copy(histw_hbm.at[pl.ds(wid * WPB, WPB)],
                  histw.at[pl.ds(0, WPB)])

  def _unpk(i, c):
    wv = histw[pl.ds(i * 16, 16)]
    counter[pl.ds(i * 16, 16)] = wv & 255
    counter[pl.ds(PLANE + i * 16, 16)] = (wv >> 8) & 255
    counter[pl.ds(2 * PLANE + i * 16, 16)] = (wv >> 16) & 255
    counter[pl.ds(3 * PLANE + i * 16, 16)] = (wv >> 24) & 255
    return c

  lax.fori_loop(0, WPB // 16, _unpk, 0)
  for p in range(4):
    pltpu.sync_copy(dense_sh.at[pl.ds(p * PLANE, PLANE)], densebuf)

    def _combo(i, c, p=p):
      dv = densebuf[pl.ds(i * 16, 16)]
      ov = counter[pl.ds(p * PLANE + i * 16, 16)]
      counter[pl.ds(p * PLANE + i * 16, 16)] = dv * 128 + ov
      return c

    lax.fori_loop(0, WPB // 16, _combo, 0)

  def _p3_win(w, c0):
    pltpu.sync_copy(pts.at[pl.ds(wid * CH + w * WROWS, WROWS), :], win)

    def _pref_i(i, c):
      idxstage[pl.ds(i * 16, 16)] = VOX_ROWS + i * 16 + iota
      return c

    lax.fori_loop(0, WVECS, _pref_i, 0)

    def _p3_vec(j, nk):
      pk = packed[pl.ds((w * WVECS + j) * 16, 16)]
      binv = lax.shift_right_logical(pk, RSH)
      lr = pk & RMASK
      combo = plsc.load_gather(counter, [(binv & 3) * PLANE + (binv >> 2)])
      rg = (combo & 127) + lr
      dnv = lax.shift_right_logical(combo, 7)
      keep = (binv < NBIN) & (rg < MAXP) & (dnv < MAXV)
      rowv = dnv * MAXP + rg
      pos = plsc.cumsum(ones, mask=keep) - 1 + nk
      plsc.store_scatter(idxstage, [pos], rowv, mask=keep)
      r = j * 16 + iota
      for c in range(C):
        g = plsc.load_gather(win, [r, zeros + c])
        plsc.store_scatter(ptstage, [pos, zeros + c],
                           plsc.bitcast(g, jnp.int32), mask=keep)
      return nk + plsc.all_reduce_population_count(keep)

    lax.fori_loop(0, WVECS, _p3_vec, zeros)
    pltpu.async_copy(ptstage, vox.at[idxstage], sem).wait()
    return c0

  lax.fori_loop(0, NWIN, _p3_win, 0)


_voxelize_sc = pl.kernel(
    _body,
    out_type=(
        jax.ShapeDtypeStruct((VOX_TOT, VC), jnp.int32),
        jax.ShapeDtypeStruct((NV_TOT,), jnp.int32),
        jax.ShapeDtypeStruct((NV_TOT, VC), jnp.int32),
        jax.ShapeDtypeStruct((NW * WPB,), jnp.int32),
    ),
    mesh=plsc.VectorSubcoreMesh(core_axis_name="c", subcore_axis_name="s",
                                num_cores=1),
    compiler_params=pltpu.CompilerParams(needs_layout_passes=False,
                                         use_tc_tiling_on_sc=False),
    scratch_types=[
        pltpu.VMEM((BINS_PAD,), jnp.int32),
        pltpu.VMEM((NW * RW,), jnp.int32),
        pltpu.VMEM((PLANE,), jnp.int32),
        pltpu.VMEM((WROWS, C), jnp.float32),
        pltpu.VMEM((CH,), jnp.int32),
        pltpu.VMEM((RB,), jnp.int32),
        pltpu.VMEM((RB,), jnp.int32),
        pltpu.VMEM((NW * 16,), jnp.int32),
        pltpu.VMEM((RB,), jnp.int32),
        pltpu.VMEM((RB,), jnp.int32),
        pltpu.VMEM((RB, VC), jnp.int32),
        pltpu.VMEM((WROWS,), jnp.int32),
        pltpu.VMEM((WROWS, VC), jnp.int32),
        pltpu.VMEM_SHARED((4 * PLANE,), jnp.int32),
        pltpu.VMEM_SHARED((NW * 16,), jnp.int32),
        pltpu.SemaphoreType.DMA,
    ],
)


@jax.jit
def kernel(points):
  pts_pad = jnp.concatenate(
      [points, jnp.full((NPAD - N, C), -1.0, jnp.float32)], axis=0)
  vox_i, nump, coors, _ = _voxelize_sc(pts_pad)
  vox = lax.bitcast_convert_type(vox_i[:VOX_ROWS, :C], jnp.float32)
  return (vox.reshape(MAXV, MAXP, C), nump[:MAXV], coors[:MAXV, :4])


# speedup vs baseline: 5.9235x; 1.0972x over previous
import jax
import jax.numpy as jnp
from jax import lax
from jax.experimental import pallas as pl
from jax.experimental.pallas import tpu as pltpu
from jax.experimental.pallas import tpu_sc as plsc

N = 200000
C = 4
NW = 16
CH = 12544
NPAD = NW * CH
NWIN = 16
WROWS = CH // NWIN
WVECS = WROWS // 16
VC = 8
NBIN = 25000
RB = 1664
BINS_PAD = NW * RB
RVECS = RB // 16
SENT = BINS_PAD - 1
WPB = BINS_PAD // 4
RW = RB // 4
RWVECS = RW // 16
PLANE = WPB
SAT = 7
RSH = 14
RMASK = (1 << RSH) - 1
MAXP = 5
MAXV = 16000
VOX_ROWS = MAXV * MAXP
VOX_TOT = 80800
VSTRIPE = VOX_TOT // NW
NV_TOT = 17664
NVSTRIPE = NV_TOT // NW


def _stream_windows(pts, wid, wina, winb, sema, semb, body, init):
  def _src(w):
    return pts.at[pl.ds(wid * CH + w * WROWS, WROWS), :]

  pltpu.async_copy(_src(0), wina, sema)

  def _outer(w2, c):
    wa = w2 * 2
    pltpu.async_copy(_src(wa + 1), winb, semb)
    pltpu.make_async_copy(_src(0), wina, sema).wait()
    c = body(wa, wina, c)

    @pl.when(w2 < NWIN // 2 - 1)
    def _():
      pltpu.async_copy(_src(wa + 2), wina, sema)

    pltpu.make_async_copy(_src(0), winb, semb).wait()
    return body(wa + 1, winb, c)

  return lax.fori_loop(0, NWIN // 2, _outer, init)


def _body(pts, vox, nump, coors, histw_hbm,
          counter, histw, densebuf, win, winb, packed, totals, vvtmp, sums2,
          numstage, densestage, coorstage, idxstage, ptstage,
          dense_sh, sums_sh, sem, sema, semb, semz):
  wid = lax.axis_index("s")
  iota = lax.iota(jnp.int32, 16)
  ones = jnp.ones((16,), jnp.int32)
  zeros = jnp.zeros((16,), jnp.int32)

  def _z_pt(i, c):
    plsc.store_scatter(ptstage, [(i * 2) + (iota >> 3), iota & 7], zeros)
    return c
  lax.fori_loop(0, WROWS * VC // 16, _z_pt, 0)

  def _z_coor(i, c):
    plsc.store_scatter(coorstage, [(i * 2) + (iota >> 3), iota & 7], zeros)
    return c
  lax.fori_loop(0, NVSTRIPE * VC // 16, _z_coor, 0)

  def _z_num(i, c):
    numstage[pl.ds(i * 16, 16)] = zeros
    return c
  lax.fori_loop(0, NVSTRIPE // 16, _z_num, 0)

  def _z_cnt(i, c):
    for u in range(4):
      counter[pl.ds((i * 4 + u) * 16, 16)] = zeros
    return c
  lax.fori_loop(0, BINS_PAD // 64, _z_cnt, 0)

  vr0 = wid * VSTRIPE
  zdmas = []
  for k in range(VSTRIPE // WROWS):
    zdmas.append(pltpu.async_copy(ptstage.at[pl.ds(0, WROWS), :],
                                  vox.at[pl.ds(vr0 + k * WROWS, WROWS), :],
                                  semz))
  rem = VSTRIPE - (VSTRIPE // WROWS) * WROWS
  zdmas.append(pltpu.async_copy(ptstage.at[pl.ds(0, rem), :],
                                vox.at[pl.ds(vr0 + VSTRIPE - rem, rem), :],
                                semz))
  nr0 = wid * NVSTRIPE
  zdmas.append(pltpu.async_copy(numstage.at[pl.ds(0, NVSTRIPE)],
                                nump.at[pl.ds(nr0, NVSTRIPE)], semz))
  zdmas.append(pltpu.async_copy(coorstage.at[pl.ds(0, NVSTRIPE), :],
                                coors.at[pl.ds(nr0, NVSTRIPE), :], semz))

  vs_xy = jnp.float32(0.02)
  vs_z = jnp.float32(0.1)

  def _p1_body(w, buf, c1):
    def _p1_vec(j, c2):
      r = j * 16 + iota
      gx = plsc.load_gather(buf, [r, zeros])
      gy = plsc.load_gather(buf, [r, zeros + 1])
      gz = plsc.load_gather(buf, [r, zeros + 2])
      vx = (gx / vs_xy).astype(jnp.int32)
      vy = (gy / vs_xy).astype(jnp.int32)
      vz = (gz / vs_z).astype(jnp.int32)
      valid = ((vx >= 0) & (vx < 50) & (vy >= 0) & (vy < 50)
               & (vz >= 0) & (vz < 10))
      binv = jnp.where(valid, vz * 2500 + vy * 50 + vx, SENT)
      cnt, lastm = plsc.scan_count(binv)
      old = plsc.load_gather(counter, [binv])
      plsc.store_scatter(counter, [binv], jnp.minimum(old + cnt, SAT),
                         mask=lastm)
      packed[pl.ds((w * WVECS + j) * 16, 16)] = (
          binv * (1 << RSH) + (old + cnt - 1))
      return c2

    lax.fori_loop(0, WVECS, _p1_vec, c1)
    return c1

  _stream_windows(pts, wid, win, winb, sema, semb, _p1_body, 0)

  def _pack(i4, c):
    for u in range(4):
      i = i4 * 4 + u
      k = (i * 16 + iota) * 4
      h0 = plsc.load_gather(counter, [k])
      h1 = plsc.load_gather(counter, [k + 1])
      h2 = plsc.load_gather(counter, [k + 2])
      h3 = plsc.load_gather(counter, [k + 3])
      histw[pl.ds(i * 16, 16)] = h0 | (h1 << 8) | (h2 << 16) | (h3 << 24)
    return c

  lax.fori_loop(0, WPB // 64, _pack, 0)
  pltpu.sync_copy(histw.at[pl.ds(0, WPB)],
                  histw_hbm.at[pl.ds(wid * WPB, WPB)])
  for d in zdmas:
    d.wait()
  plsc.subcore_barrier()

  w0 = wid * RW
  hdmas = [pltpu.async_copy(histw_hbm.at[pl.ds(wp * WPB + w0, RW)],
                            histw.at[pl.ds(wp * RW, RW)], sem)
           for wp in range(NW)]
  for d in hdmas:
    d.wait()

  def _p2a(i, c):
    acc0 = zeros
    acc1 = zeros
    acc2 = zeros
    acc3 = zeros
    for wp in range(NW):
      wv = histw[pl.ds(wp * RW + i * 16, 16)]
      h0 = wv & 255
      h1 = (wv >> 8) & 255
      h2 = (wv >> 16) & 255
      h3 = (wv >> 24) & 255
      histw[pl.ds(wp * RW + i * 16, 16)] = (
          acc0 | (acc1 << 8) | (acc2 << 16) | (acc3 << 24))
      acc0 = acc0 + h0
      acc1 = acc1 + h1
      acc2 = acc2 + h2
      acc3 = acc3 + h3
    k = (i * 16 + iota) * 4
    plsc.store_scatter(totals, [k], acc0)
    plsc.store_scatter(totals, [k + 1], acc1)
    plsc.store_scatter(totals, [k + 2], acc2)
    plsc.store_scatter(totals, [k + 3], acc3)
    return c

  lax.fori_loop(0, RWVECS, _p2a, 0)
  hdmas = [pltpu.async_copy(histw.at[pl.ds(wp * RW, RW)],
                            histw_hbm.at[pl.ds(wp * WPB + w0, RW)], sem)
           for wp in range(NW)]
  for d in hdmas:
    d.wait()

  col0 = wid * RB

  def _p2b(j, carry):
    tv = totals[pl.ds(j * 16, 16)]
    binv = col0 + j * 16 + iota
    occ = (tv > 0) & (binv < NBIN)
    cinc = plsc.cumsum(ones, mask=occ) + carry
    vvtmp[pl.ds(j * 16, 16)] = cinc
    return jnp.max(cinc)

  range_occ = lax.fori_loop(0, RVECS, _p2b, jnp.int32(0))
  sums2[pl.ds(0, 16)] = zeros + range_occ
  pltpu.sync_copy(sums2.at[pl.ds(0, 16)], sums_sh.at[pl.ds(wid * 16, 16)])
  plsc.subcore_barrier()

  pltpu.sync_copy(sums_sh, sums2)
  svec = plsc.load_gather(sums2, [iota * 16 + iota])
  base = jnp.sum(jnp.where(iota < wid, svec, 0))

  def _pref_d(i, c):
    densestage[pl.ds(i * 16, 16)] = MAXV + i * 16 + iota
    return c

  lax.fori_loop(0, RVECS, _pref_d, 0)

  def _p2d(j, nk):
    cinc = vvtmp[pl.ds(j * 16, 16)]
    tv = totals[pl.ds(j * 16, 16)]
    lbl = j * 16 + iota
    binv = col0 + lbl
    occ = (tv > 0) & (binv < NBIN)
    dnv = base + cinc - occ.astype(jnp.int32)
    plsc.store_scatter(histw, [(lbl & 3) * RW + (lbl >> 2)], dnv)
    keep = occ & (dnv < MAXV)
    pos = plsc.cumsum(ones, mask=keep) - 1 + nk
    numv = jnp.minimum(tv, MAXP)
    z = lax.div(binv, 2500)
    rm = binv - z * 2500
    y = lax.div(rm, 50)
    x = rm - y * 50
    plsc.store_scatter(numstage, [pos], numv, mask=keep)
    plsc.store_scatter(densestage, [pos], dnv, mask=keep)
    plsc.store_scatter(coorstage, [pos, zeros], zeros, mask=keep)
    plsc.store_scatter(coorstage, [pos, zeros + 1], z, mask=keep)
    plsc.store_scatter(coorstage, [pos, zeros + 2], y, mask=keep)
    plsc.store_scatter(coorstage, [pos, zeros + 3], x, mask=keep)
    return nk + plsc.all_reduce_population_count(keep)

  lax.fori_loop(0, RVECS, _p2d, zeros)
  for p in range(4):
    pltpu.sync_copy(histw.at[pl.ds(p * RW, RW)],
                    dense_sh.at[pl.ds(p * PLANE + wid * RW, RW)])
  pltpu.async_copy(numstage, nump.at[densestage], sem).wait()
  pltpu.async_copy(coorstage, coors.at[densestage], sem).wait()
  plsc.subcore_barrier()

  pltpu.sync_copy(histw_hbm.at[pl.ds(wid * WPB, WPB)],
                  histw.at[pl.ds(0, WPB)])

  def _unpk(i4, c):
    for u in range(4):
      i = i4 * 4 + u
      wv = histw[pl.ds(i * 16, 16)]
      counter[pl.ds(i * 16, 16)] = wv & 255
      counter[pl.ds(PLANE + i * 16, 16)] = (wv >> 8) & 255
      counter[pl.ds(2 * PLANE + i * 16, 16)] = (wv >> 16) & 255
      counter[pl.ds(3 * PLANE + i * 16, 16)] = (wv >> 24) & 255
    return c

  lax.fori_loop(0, WPB // 64, _unpk, 0)
  for p in range(4):
    pltpu.sync_copy(dense_sh.at[pl.ds(p * PLANE, PLANE)], densebuf)

    def _combo(i4, c, p=p):
      for u in range(4):
        i = i4 * 4 + u
        dv = densebuf[pl.ds(i * 16, 16)]
        ov = counter[pl.ds(p * PLANE + i * 16, 16)]
        counter[pl.ds(p * PLANE + i * 16, 16)] = dv * 128 + ov
      return c

    lax.fori_loop(0, WPB // 64, _combo, 0)

  def _p3_body(w, buf, c1):
    def _pref_i(i, c):
      idxstage[pl.ds(i * 16, 16)] = VOX_ROWS + i * 16 + iota
      return c

    lax.fori_loop(0, WVECS, _pref_i, 0)

    @plsc.parallel_loop(0, WVECS, carry=zeros)
    def _p3_vec(j, nk):
      pk = packed[pl.ds((w * WVECS + j) * 16, 16)]
      binv = lax.shift_right_logical(pk, RSH)
      lr = pk & RMASK
      combo = plsc.load_gather(counter, [(binv & 3) * PLANE + (binv >> 2)])
      rg = (combo & 127) + lr
      dnv = lax.shift_right_logical(combo, 7)
      keep = (binv < NBIN) & (rg < MAXP) & (dnv < MAXV)
      rowv = dnv * MAXP + rg
      pos = plsc.cumsum(ones, mask=keep) - 1 + nk
      plsc.store_scatter(idxstage, [pos], rowv, mask=keep)
      r = j * 16 + iota
      for c in range(C):
        g = plsc.load_gather(buf, [r, zeros + c])
        plsc.store_scatter(ptstage, [pos, zeros + c],
                           plsc.bitcast(g, jnp.int32), mask=keep)
      return nk + plsc.all_reduce_population_count(keep)

    pltpu.async_copy(ptstage, vox.at[idxstage], sem).wait()
    return c1

  _stream_windows(pts, wid, win, winb, sema, semb, _p3_body, 0)


_voxelize_sc = pl.kernel(
    _body,
    out_type=(
        jax.ShapeDtypeStruct((VOX_TOT, VC), jnp.int32),
        jax.ShapeDtypeStruct((NV_TOT,), jnp.int32),
        jax.ShapeDtypeStruct((NV_TOT, VC), jnp.int32),
        jax.ShapeDtypeStruct((NW * WPB,), jnp.int32),
    ),
    mesh=plsc.VectorSubcoreMesh(core_axis_name="c", subcore_axis_name="s",
                                num_cores=1),
    compiler_params=pltpu.CompilerParams(needs_layout_passes=False,
                                         use_tc_tiling_on_sc=False),
    scratch_types=[
        pltpu.VMEM((BINS_PAD,), jnp.int32),
        pltpu.VMEM((NW * RW,), jnp.int32),
        pltpu.VMEM((PLANE,), jnp.int32),
        pltpu.VMEM((WROWS, C), jnp.float32),
        pltpu.VMEM((WROWS, C), jnp.float32),
        pltpu.VMEM((CH,), jnp.int32),
        pltpu.VMEM((RB,), jnp.int32),
        pltpu.VMEM((RB,), jnp.int32),
        pltpu.VMEM((NW * 16,), jnp.int32),
        pltpu.VMEM((RB,), jnp.int32),
        pltpu.VMEM((RB,), jnp.int32),
        pltpu.VMEM((RB, VC), jnp.int32),
        pltpu.VMEM((WROWS,), jnp.int32),
        pltpu.VMEM((WROWS, VC), jnp.int32),
        pltpu.VMEM_SHARED((4 * PLANE,), jnp.int32),
        pltpu.VMEM_SHARED((NW * 16,), jnp.int32),
        pltpu.SemaphoreType.DMA,
        pltpu.SemaphoreType.DMA,
        pltpu.SemaphoreType.DMA,
        pltpu.SemaphoreType.DMA,
    ],
)


@jax.jit
def kernel(points):
  pts_pad = jnp.concatenate(
      [points, jnp.full((NPAD - N, C), -1.0, jnp.float32)], axis=0)
  vox_i, nump, coors, _ = _voxelize_sc(pts_pad)
  vox = lax.bitcast_convert_type(vox_i[:VOX_ROWS, :C], jnp.float32)
  return (vox.reshape(MAXV, MAXP, C), nump[:MAXV], coors[:MAXV, :4])


# speedup vs baseline: 5.9369x; 1.0023x over previous
import jax
import jax.numpy as jnp
from jax import lax
from jax.experimental import pallas as pl
from jax.experimental.pallas import tpu as pltpu
from jax.experimental.pallas import tpu_sc as plsc

N = 200000
C = 4
NW = 16
CH = 12544
NPAD = NW * CH
NWIN = 16
WROWS = CH // NWIN
WVECS = WROWS // 16
VC = 8
NBIN = 25000
RB = 1664
BINS_PAD = NW * RB
RVECS = RB // 16
SENT = BINS_PAD - 1
WPB = BINS_PAD // 4
RW = RB // 4
RWVECS = RW // 16
PLANE = WPB
SAT = 7
RSH = 14
RMASK = (1 << RSH) - 1
MAXP = 5
MAXV = 16000
VOX_ROWS = MAXV * MAXP
VOX_TOT = VOX_ROWS + CH
VSTRIPE = VOX_ROWS // NW
NV_TOT = 17664
NVSTRIPE = NV_TOT // NW


def _stream_windows(pts, wid, wina, winb, sema, semb, body, init):
  def _src(w):
    return pts.at[pl.ds(wid * CH + w * WROWS, WROWS), :]

  pltpu.async_copy(_src(0), wina, sema)

  def _outer(w2, c):
    wa = w2 * 2
    pltpu.async_copy(_src(wa + 1), winb, semb)
    pltpu.make_async_copy(_src(0), wina, sema).wait()
    c = body(wa, wina, c)

    @pl.when(w2 < NWIN // 2 - 1)
    def _():
      pltpu.async_copy(_src(wa + 2), wina, sema)

    pltpu.make_async_copy(_src(0), winb, semb).wait()
    return body(wa + 1, winb, c)

  return lax.fori_loop(0, NWIN // 2, _outer, init)


def _body(pts, vox, nump, coors, histw_hbm,
          counter, histw, densebuf, win, winb, packed, totals, vvtmp, sums2,
          numstage, densestage, coorstage, idxstage, ptstage,
          dense_sh, sums_sh, sem, sema, semb, semz):
  wid = lax.axis_index("s")
  iota = lax.iota(jnp.int32, 16)
  ones = jnp.ones((16,), jnp.int32)
  zeros = jnp.zeros((16,), jnp.int32)

  fzeros = jnp.zeros((16,), jnp.float32)

  def _z_pt(i, c):
    plsc.store_scatter(ptstage, [(i * 2) + (iota >> 3), iota & 7], fzeros)
    return c
  lax.fori_loop(0, WROWS * VC // 16, _z_pt, 0)

  def _z_coor(i, c):
    plsc.store_scatter(coorstage, [(i * 2) + (iota >> 3), iota & 7], zeros)
    return c
  lax.fori_loop(0, NVSTRIPE * VC // 16, _z_coor, 0)

  def _z_num(i, c):
    numstage[pl.ds(i * 16, 16)] = zeros
    return c
  lax.fori_loop(0, NVSTRIPE // 16, _z_num, 0)

  def _z_cnt(i, c):
    for u in range(4):
      counter[pl.ds((i * 4 + u) * 16, 16)] = zeros
    return c
  lax.fori_loop(0, BINS_PAD // 64, _z_cnt, 0)

  vr0 = wid * VSTRIPE
  zdmas = []
  for k in range(VSTRIPE // WROWS):
    zdmas.append(pltpu.async_copy(ptstage.at[pl.ds(0, WROWS), :],
                                  vox.at[pl.ds(vr0 + k * WROWS, WROWS), :],
                                  semz))
  rem = VSTRIPE - (VSTRIPE // WROWS) * WROWS
  zdmas.append(pltpu.async_copy(ptstage.at[pl.ds(0, rem), :],
                                vox.at[pl.ds(vr0 + VSTRIPE - rem, rem), :],
                                semz))
  nr0 = wid * NVSTRIPE
  zdmas.append(pltpu.async_copy(numstage.at[pl.ds(0, NVSTRIPE)],
                                nump.at[pl.ds(nr0, NVSTRIPE)], semz))
  zdmas.append(pltpu.async_copy(coorstage.at[pl.ds(0, NVSTRIPE), :],
                                coors.at[pl.ds(nr0, NVSTRIPE), :], semz))

  vs_xy = jnp.float32(0.02)
  vs_z = jnp.float32(0.1)

  def _p1_body(w, buf, c1):
    def _p1_vec(j, c2):
      r = j * 16 + iota
      gx = plsc.load_gather(buf, [r, zeros])
      gy = plsc.load_gather(buf, [r, zeros + 1])
      gz = plsc.load_gather(buf, [r, zeros + 2])
      vx = (gx / vs_xy).astype(jnp.int32)
      vy = (gy / vs_xy).astype(jnp.int32)
      vz = (gz / vs_z).astype(jnp.int32)
      valid = ((vx >= 0) & (vx < 50) & (vy >= 0) & (vy < 50)
               & (vz >= 0) & (vz < 10))
      binv = jnp.where(valid, vz * 2500 + vy * 50 + vx, SENT)
      cnt, lastm = plsc.scan_count(binv)
      old = plsc.load_gather(counter, [binv])
      plsc.store_scatter(counter, [binv], jnp.minimum(old + cnt, SAT),
                         mask=lastm)
      packed[pl.ds((w * WVECS + j) * 16, 16)] = (
          binv * (1 << RSH) + (old + cnt - 1))
      return c2

    lax.fori_loop(0, WVECS, _p1_vec, c1)
    return c1

  _stream_windows(pts, wid, win, winb, sema, semb, _p1_body, 0)

  def _pack(i4, c):
    for u in range(4):
      i = i4 * 4 + u
      k = (i * 16 + iota) * 4
      h0 = plsc.load_gather(counter, [k])
      h1 = plsc.load_gather(counter, [k + 1])
      h2 = plsc.load_gather(counter, [k + 2])
      h3 = plsc.load_gather(counter, [k + 3])
      histw[pl.ds(i * 16, 16)] = h0 | (h1 << 8) | (h2 << 16) | (h3 << 24)
    return c

  lax.fori_loop(0, WPB // 64, _pack, 0)
  pltpu.sync_copy(histw.at[pl.ds(0, WPB)],
                  histw_hbm.at[pl.ds(wid * WPB, WPB)])
  for d in zdmas:
    d.wait()
  plsc.subcore_barrier()

  w0 = wid * RW
  hdmas = [pltpu.async_copy(histw_hbm.at[pl.ds(wp * WPB + w0, RW)],
                            histw.at[pl.ds(wp * RW, RW)], sem)
           for wp in range(NW)]
  for d in hdmas:
    d.wait()

  def _p2a(i, c):
    acc0 = zeros
    acc1 = zeros
    acc2 = zeros
    acc3 = zeros
    for wp in range(NW):
      wv = histw[pl.ds(wp * RW + i * 16, 16)]
      h0 = wv & 255
      h1 = (wv >> 8) & 255
      h2 = (wv >> 16) & 255
      h3 = (wv >> 24) & 255
      histw[pl.ds(wp * RW + i * 16, 16)] = (
          acc0 | (acc1 << 8) | (acc2 << 16) | (acc3 << 24))
      acc0 = acc0 + h0
      acc1 = acc1 + h1
      acc2 = acc2 + h2
      acc3 = acc3 + h3
    k = (i * 16 + iota) * 4
    plsc.store_scatter(totals, [k], acc0)
    plsc.store_scatter(totals, [k + 1], acc1)
    plsc.store_scatter(totals, [k + 2], acc2)
    plsc.store_scatter(totals, [k + 3], acc3)
    return c

  lax.fori_loop(0, RWVECS, _p2a, 0)
  hdmas = [pltpu.async_copy(histw.at[pl.ds(wp * RW, RW)],
                            histw_hbm.at[pl.ds(wp * WPB + w0, RW)], sem)
           for wp in range(NW)]
  for d in hdmas:
    d.wait()

  col0 = wid * RB

  def _p2b(j, carry):
    tv = totals[pl.ds(j * 16, 16)]
    binv = col0 + j * 16 + iota
    occ = (tv > 0) & (binv < NBIN)
    cinc = plsc.cumsum(ones, mask=occ) + carry
    vvtmp[pl.ds(j * 16, 16)] = cinc
    return jnp.max(cinc)

  range_occ = lax.fori_loop(0, RVECS, _p2b, jnp.int32(0))
  sums2[pl.ds(0, 16)] = zeros + range_occ
  pltpu.sync_copy(sums2.at[pl.ds(0, 16)], sums_sh.at[pl.ds(wid * 16, 16)])
  plsc.subcore_barrier()

  pltpu.sync_copy(sums_sh, sums2)
  svec = plsc.load_gather(sums2, [iota * 16 + iota])
  base = jnp.sum(jnp.where(iota < wid, svec, 0))

  def _p2d(j, c):
    cinc = vvtmp[pl.ds(j * 16, 16)]
    tv = totals[pl.ds(j * 16, 16)]
    lbl = j * 16 + iota
    binv = col0 + lbl
    occ = (tv > 0) & (binv < NBIN)
    dnv = base + cinc - occ.astype(jnp.int32)
    plsc.store_scatter(histw, [(lbl & 3) * RW + (lbl >> 2)], dnv)
    keep = occ & (dnv < MAXV)
    numv = jnp.minimum(tv, MAXP)
    z = lax.div(binv, 2500)
    rm = binv - z * 2500
    y = lax.div(rm, 50)
    x = rm - y * 50
    densestage[pl.ds(j * 16, 16)] = jnp.where(keep, dnv, MAXV + lbl)
    numstage[pl.ds(j * 16, 16)] = numv
    plsc.store_scatter(coorstage, [lbl, zeros], zeros)
    plsc.store_scatter(coorstage, [lbl, zeros + 1], z)
    plsc.store_scatter(coorstage, [lbl, zeros + 2], y)
    plsc.store_scatter(coorstage, [lbl, zeros + 3], x)
    return c

  lax.fori_loop(0, RVECS, _p2d, 0)
  for p in range(4):
    pltpu.sync_copy(histw.at[pl.ds(p * RW, RW)],
                    dense_sh.at[pl.ds(p * PLANE + wid * RW, RW)])
  pltpu.async_copy(numstage, nump.at[densestage], sem).wait()
  pltpu.async_copy(coorstage, coors.at[densestage], sem).wait()
  plsc.subcore_barrier()

  pltpu.sync_copy(histw_hbm.at[pl.ds(wid * WPB, WPB)],
                  histw.at[pl.ds(0, WPB)])

  for p in range(4):
    pltpu.sync_copy(dense_sh.at[pl.ds(p * PLANE, PLANE)], densebuf)

    def _combo(i4, c, p=p):
      for u in range(4):
        i = i4 * 4 + u
        wv = histw[pl.ds(i * 16, 16)]
        dv = densebuf[pl.ds(i * 16, 16)]
        counter[pl.ds(p * PLANE + i * 16, 16)] = (
            dv * 128 + ((wv >> (8 * p)) & 255))
      return c

    lax.fori_loop(0, WPB // 64, _combo, 0)

  def _p3_body(w, buf, c1):
    @plsc.parallel_loop(0, WVECS)
    def _p3_vec(j):
      pk = packed[pl.ds((w * WVECS + j) * 16, 16)]
      binv = lax.shift_right_logical(pk, RSH)
      lr = pk & RMASK
      combo = plsc.load_gather(counter, [(binv & 3) * PLANE + (binv >> 2)])
      rg = (combo & 127) + lr
      dnv = lax.shift_right_logical(combo, 7)
      keep = (binv < NBIN) & (rg < MAXP) & (dnv < MAXV)
      idxstage[pl.ds(j * 16, 16)] = jnp.where(
          keep, dnv * MAXP + rg, VOX_ROWS + (w * WVECS + j) * 16 + iota)

    pltpu.async_copy(buf, vox.at[idxstage], sem).wait()
    return c1

  _stream_windows(pts, wid, win, winb, sema, semb, _p3_body, 0)


_voxelize_sc = pl.kernel(
    _body,
    out_type=(
        jax.ShapeDtypeStruct((VOX_TOT, VC), jnp.float32),
        jax.ShapeDtypeStruct((NV_TOT,), jnp.int32),
        jax.ShapeDtypeStruct((NV_TOT, VC), jnp.int32),
        jax.ShapeDtypeStruct((NW * WPB,), jnp.int32),
    ),
    mesh=plsc.VectorSubcoreMesh(core_axis_name="c", subcore_axis_name="s",
                                num_cores=1),
    compiler_params=pltpu.CompilerParams(needs_layout_passes=False,
                                         use_tc_tiling_on_sc=False),
    scratch_types=[
        pltpu.VMEM((BINS_PAD,), jnp.int32),
        pltpu.VMEM((NW * RW,), jnp.int32),
        pltpu.VMEM((PLANE,), jnp.int32),
        pltpu.VMEM((WROWS, VC), jnp.float32),
        pltpu.VMEM((WROWS, VC), jnp.float32),
        pltpu.VMEM((CH,), jnp.int32),
        pltpu.VMEM((RB,), jnp.int32),
        pltpu.VMEM((RB,), jnp.int32),
        pltpu.VMEM((NW * 16,), jnp.int32),
        pltpu.VMEM((RB,), jnp.int32),
        pltpu.VMEM((RB,), jnp.int32),
        pltpu.VMEM((RB, VC), jnp.int32),
        pltpu.VMEM((WROWS,), jnp.int32),
        pltpu.VMEM((WROWS, VC), jnp.float32),
        pltpu.VMEM_SHARED((4 * PLANE,), jnp.int32),
        pltpu.VMEM_SHARED((NW * 16,), jnp.int32),
        pltpu.SemaphoreType.DMA,
        pltpu.SemaphoreType.DMA,
        pltpu.SemaphoreType.DMA,
        pltpu.SemaphoreType.DMA,
    ],
)


@jax.jit
def kernel(points):
  pts_pad = jnp.pad(points, ((0, NPAD - N), (0, VC - C)),
                    constant_values=-1.0)
  vox8, nump, coors, _ = _voxelize_sc(pts_pad)
  return (vox8[:VOX_ROWS, :C].reshape(MAXV, MAXP, C),
          nump[:MAXV], coors[:MAXV, :4])
